# baseline (speedup 1.0000x reference)
import jax
import jax.numpy as jnp
from jax import lax
from jax.experimental import pallas as pl
from jax.experimental.pallas import tpu as pltpu
from jax.experimental.pallas import tpu_sc as plsc

_TILE = 2048
_EPS = 1e-6


def _tc_body(x_ref, y_ref, xnT_ref, ynT_ref,
             dist_ref, idxx_ref, idxy_ref, xh_ref, yh_ref):
    n = pl.program_id(0)
    nbatch = pl.num_programs(0)
    P1 = x_ref.shape[1]
    P2 = y_ref.shape[1]
    nt = P2 // _TILE

    x = x_ref[0]
    x2 = jnp.sum(x * x, axis=1, keepdims=True)

    xnT = xnT_ref[0]
    ynT = ynT_ref[0]
    xh_ref[0] = xnT / jnp.maximum(
        jnp.sqrt(jnp.sum(xnT * xnT, axis=0, keepdims=True)), _EPS)
    yh_ref[0] = ynT / jnp.maximum(
        jnp.sqrt(jnp.sum(ynT * ynT, axis=0, keepdims=True)), _EPS)

    big = jnp.float32(3.4e38)
    bigi = jnp.int32(2**30)

    def tile_step(t, carry):
        run_min, run_idx, s_cham_y = carry
        yt = y_ref[0, pl.ds(t * _TILE, _TILE), :]
        y2t = jnp.sum(yt * yt, axis=1)[None, :]
        xy = lax.dot_general(x, yt, (((1,), (1,)), ((), ())),
                             preferred_element_type=jnp.float32)
        d = x2 + y2t - 2.0 * xy

        riota = lax.broadcasted_iota(jnp.int32, (P1, _TILE), 0)
        cmin = jnp.min(d, axis=0, keepdims=True)
        cidx = jnp.min(jnp.where(d == cmin, riota, bigi),
                       axis=0, keepdims=True)
        idxy_ref[0, 0, pl.ds(t * _TILE, _TILE)] = cidx[0]
        s_cham_y = s_cham_y + jnp.sum(cmin)

        ciota = lax.broadcasted_iota(jnp.int32, (P1, _TILE), 1)
        tmin = jnp.min(d, axis=1, keepdims=True)
        tidx = jnp.min(jnp.where(d == tmin, ciota, bigi),
                       axis=1, keepdims=True) + t * _TILE
        better = tmin < run_min
        run_min = jnp.where(better, tmin, run_min)
        run_idx = jnp.where(better, tidx, run_idx)
        return run_min, run_idx, s_cham_y

    run_min, run_idx, s_cham_y = lax.fori_loop(
        0, nt, tile_step,
        (jnp.full((P1, 1), big, jnp.float32),
         jnp.zeros((P1, 1), jnp.int32),
         jnp.float32(0.0)))

    idxx_ref[0] = run_idx

    s_cham_x = jnp.sum(run_min)
    d_contrib = (s_cham_x / P1 + s_cham_y / P2) / nbatch

    @pl.when(n == 0)
    def _init():
        dist_ref[...] = jnp.zeros((1, 1), jnp.float32)

    dist_ref[...] += d_contrib.reshape(1, 1)


def _sc_body(xh_hbm, yh_hbm, idxx_hbm, idxy_hbm, out_hbm,
             xh_v, yh_v, ix_v, iy_v, acc_v):
    c = lax.axis_index("c")
    s = lax.axis_index("s")
    wid = s * 2 + c
    b = wid // 8
    base = (wid % 8) * 256
    P = 2048

    pltpu.sync_copy(xh_hbm.at[b], xh_v)
    pltpu.sync_copy(yh_hbm.at[b], yh_v)
    pltpu.sync_copy(idxx_hbm.at[b, pl.ds(base, 256)], ix_v)
    pltpu.sync_copy(idxy_hbm.at[b, pl.ds(base, 256)], iy_v)

    hi = jnp.full((16,), P - 1, jnp.int32)

    def step(i, acc):
        ix = jnp.minimum(ix_v[pl.ds(i * 16, 16)], hi)
        iy = jnp.minimum(iy_v[pl.ds(i * 16, 16)], hi)
        cx = jnp.zeros((16,), jnp.float32)
        cy = jnp.zeros((16,), jnp.float32)
        for k in range(3):
            gx = plsc.load_gather(yh_v, [ix + k * P])
            gy = plsc.load_gather(xh_v, [iy + k * P])
            ax = xh_v[pl.ds(k * P + base + i * 16, 16)]
            ay = yh_v[pl.ds(k * P + base + i * 16, 16)]
            cx = cx + ax * gx
            cy = cy + ay * gy
        return acc + (2.0 - jnp.abs(cx) - jnp.abs(cy))

    acc = lax.fori_loop(0, 16, step, jnp.zeros((16,), jnp.float32))
    acc_v[...] = acc
    pltpu.sync_copy(acc_v, out_hbm.at[wid])


def _sc_normals(xh, yh, idxx, idxy):
    fn = pl.kernel(
        _sc_body,
        out_type=jax.ShapeDtypeStruct((32, 16), jnp.float32),
        mesh=plsc.VectorSubcoreMesh(core_axis_name="c", subcore_axis_name="s"),
        scratch_types=[
            pltpu.VMEM((3 * 2048,), jnp.float32),
            pltpu.VMEM((3 * 2048,), jnp.float32),
            pltpu.VMEM((256,), jnp.int32),
            pltpu.VMEM((256,), jnp.int32),
            pltpu.VMEM((16,), jnp.float32),
        ],
        compiler_params=pltpu.CompilerParams(needs_layout_passes=False),
    )
    return fn(xh, yh, idxx, idxy)


def kernel(x, y, x_normals, y_normals):
    N, P1, D = x.shape
    P2 = y.shape[1]
    xnT = jnp.transpose(x_normals, (0, 2, 1))
    ynT = jnp.transpose(y_normals, (0, 2, 1))
    H = N // 2

    def tc_half(xs, ys, xnTs, ynTs):
        return pl.pallas_call(
            _tc_body,
            grid=(H,),
            in_specs=[
                pl.BlockSpec((1, P1, D), lambda n: (n, 0, 0)),
                pl.BlockSpec((1, P2, D), lambda n: (n, 0, 0)),
                pl.BlockSpec((1, D, P1), lambda n: (n, 0, 0)),
                pl.BlockSpec((1, D, P2), lambda n: (n, 0, 0)),
            ],
            out_specs=[
                pl.BlockSpec((1, 1), lambda n: (0, 0)),
                pl.BlockSpec((1, P1, 1), lambda n: (n, 0, 0)),
                pl.BlockSpec((1, 1, P2), lambda n: (n, 0, 0)),
                pl.BlockSpec((1, D, P1), lambda n: (n, 0, 0)),
                pl.BlockSpec((1, D, P2), lambda n: (n, 0, 0)),
            ],
            out_shape=[
                jax.ShapeDtypeStruct((1, 1), jnp.float32),
                jax.ShapeDtypeStruct((H, P1, 1), jnp.int32),
                jax.ShapeDtypeStruct((H, 1, P2), jnp.int32),
                jax.ShapeDtypeStruct((H, D, P1), jnp.float32),
                jax.ShapeDtypeStruct((H, D, P2), jnp.float32),
            ],
        )(xs, ys, xnTs, ynTs)

    d1, ix1, iy1, xh1, yh1 = tc_half(x[:H], y[:H], xnT[:H], ynT[:H])
    p1 = _sc_normals(xh1.reshape(H, D * P1), yh1.reshape(H, D * P2),
                     ix1.reshape(H, P1), iy1.reshape(H, P2))
    d2, ix2, iy2, xh2, yh2 = tc_half(x[H:], y[H:], xnT[H:], ynT[H:])
    p2 = _sc_normals(xh2.reshape(H, D * P1), yh2.reshape(H, D * P2),
                     ix2.reshape(H, P1), iy2.reshape(H, P2))
    cham_dist = (d1[0, 0] + d2[0, 0]) / 2.0
    cham_normals = (jnp.sum(p1) + jnp.sum(p2)) / (P1 * N)
    return (cham_dist, cham_normals)


# speedup vs baseline: 1.1217x; 1.1217x over previous
import jax
import jax.numpy as jnp
from jax import lax
from jax.experimental import pallas as pl
from jax.experimental.pallas import tpu as pltpu
from jax.experimental.pallas import tpu_sc as plsc

_TILE = 2048
_EPS = 1e-6


def _tc_body(x_ref, y_ref, xnT_ref, ynT_ref,
             dist_ref, idxx_ref, idxy_ref, xh_ref, yh_ref):
    n = pl.program_id(0)
    nbatch = pl.num_programs(0)
    P1 = x_ref.shape[1]
    P2 = y_ref.shape[1]
    nt = P2 // _TILE

    x = x_ref[0]
    x2 = jnp.sum(x * x, axis=1, keepdims=True)

    xnT = xnT_ref[0]
    ynT = ynT_ref[0]
    xh_ref[0] = xnT / jnp.maximum(
        jnp.sqrt(jnp.sum(xnT * xnT, axis=0, keepdims=True)), _EPS)
    yh_ref[0] = ynT / jnp.maximum(
        jnp.sqrt(jnp.sum(ynT * ynT, axis=0, keepdims=True)), _EPS)

    big = jnp.float32(3.4e38)
    bigi = jnp.int32(2**30)

    def tile_step(t, carry):
        run_min, run_idx, s_cham_y = carry
        yt = y_ref[0, pl.ds(t * _TILE, _TILE), :]
        y2t = jnp.sum(yt * yt, axis=1)[None, :]
        xy = lax.dot_general(x, yt, (((1,), (1,)), ((), ())),
                             preferred_element_type=jnp.float32)
        d = x2 + y2t - 2.0 * xy

        riota = lax.broadcasted_iota(jnp.int32, (P1, _TILE), 0)
        cmin = jnp.min(d, axis=0, keepdims=True)
        cidx = jnp.min(jnp.where(d == cmin, riota, bigi),
                       axis=0, keepdims=True)
        idxy_ref[0, 0, pl.ds(t * _TILE, _TILE)] = cidx[0]
        s_cham_y = s_cham_y + jnp.sum(cmin)

        ciota = lax.broadcasted_iota(jnp.int32, (P1, _TILE), 1)
        tmin = jnp.min(d, axis=1, keepdims=True)
        tidx = jnp.min(jnp.where(d == tmin, ciota, bigi),
                       axis=1, keepdims=True) + t * _TILE
        better = tmin < run_min
        run_min = jnp.where(better, tmin, run_min)
        run_idx = jnp.where(better, tidx, run_idx)
        return run_min, run_idx, s_cham_y

    run_min, run_idx, s_cham_y = lax.fori_loop(
        0, nt, tile_step,
        (jnp.full((P1, 1), big, jnp.float32),
         jnp.zeros((P1, 1), jnp.int32),
         jnp.float32(0.0)))

    idxx_ref[0] = run_idx

    s_cham_x = jnp.sum(run_min)
    d_contrib = (s_cham_x / P1 + s_cham_y / P2) / nbatch

    @pl.when(n == 0)
    def _init():
        dist_ref[...] = jnp.zeros((1, 1), jnp.float32)

    dist_ref[...] += d_contrib.reshape(1, 1)


def _sc_body(xh_hbm, yh_hbm, idxx_hbm, idxy_hbm, out_hbm,
             xh_v, yh_v, ix_v, iy_v, acc_v):
    c = lax.axis_index("c")
    s = lax.axis_index("s")
    wid = s * 2 + c
    b = wid // 4
    base = (wid % 4) * 512
    P = 2048

    pltpu.sync_copy(xh_hbm.at[b], xh_v)
    pltpu.sync_copy(yh_hbm.at[b], yh_v)
    pltpu.sync_copy(idxx_hbm.at[b, pl.ds(base, 512)], ix_v)
    pltpu.sync_copy(idxy_hbm.at[b, pl.ds(base, 512)], iy_v)

    hi = jnp.full((16,), P - 1, jnp.int32)

    def step(i, acc):
        ix = jnp.minimum(ix_v[pl.ds(i * 16, 16)], hi)
        iy = jnp.minimum(iy_v[pl.ds(i * 16, 16)], hi)
        cx = jnp.zeros((16,), jnp.float32)
        cy = jnp.zeros((16,), jnp.float32)
        for k in range(3):
            gx = plsc.load_gather(yh_v, [ix + k * P])
            gy = plsc.load_gather(xh_v, [iy + k * P])
            ax = xh_v[pl.ds(k * P + base + i * 16, 16)]
            ay = yh_v[pl.ds(k * P + base + i * 16, 16)]
            cx = cx + ax * gx
            cy = cy + ay * gy
        return acc + (2.0 - jnp.abs(cx) - jnp.abs(cy))

    acc = lax.fori_loop(0, 32, step, jnp.zeros((16,), jnp.float32))
    acc_v[...] = acc
    pltpu.sync_copy(acc_v, out_hbm.at[wid])


def _sc_normals(xh, yh, idxx, idxy):
    fn = pl.kernel(
        _sc_body,
        out_type=jax.ShapeDtypeStruct((32, 16), jnp.float32),
        mesh=plsc.VectorSubcoreMesh(core_axis_name="c", subcore_axis_name="s"),
        scratch_types=[
            pltpu.VMEM((3 * 2048,), jnp.float32),
            pltpu.VMEM((3 * 2048,), jnp.float32),
            pltpu.VMEM((512,), jnp.int32),
            pltpu.VMEM((512,), jnp.int32),
            pltpu.VMEM((16,), jnp.float32),
        ],
        compiler_params=pltpu.CompilerParams(needs_layout_passes=False),
    )
    return fn(xh, yh, idxx, idxy)


def kernel(x, y, x_normals, y_normals):
    N, P1, D = x.shape
    P2 = y.shape[1]
    xnT = jnp.transpose(x_normals, (0, 2, 1))
    ynT = jnp.transpose(y_normals, (0, 2, 1))
    dist, idxx, idxy, xh, yh = pl.pallas_call(
        _tc_body,
        grid=(N,),
        in_specs=[
            pl.BlockSpec((1, P1, D), lambda n: (n, 0, 0)),
            pl.BlockSpec((1, P2, D), lambda n: (n, 0, 0)),
            pl.BlockSpec((1, D, P1), lambda n: (n, 0, 0)),
            pl.BlockSpec((1, D, P2), lambda n: (n, 0, 0)),
        ],
        out_specs=[
            pl.BlockSpec((1, 1), lambda n: (0, 0)),
            pl.BlockSpec((1, P1, 1), lambda n: (n, 0, 0)),
            pl.BlockSpec((1, 1, P2), lambda n: (n, 0, 0)),
            pl.BlockSpec((1, D, P1), lambda n: (n, 0, 0)),
            pl.BlockSpec((1, D, P2), lambda n: (n, 0, 0)),
        ],
        out_shape=[
            jax.ShapeDtypeStruct((1, 1), jnp.float32),
            jax.ShapeDtypeStruct((N, P1, 1), jnp.int32),
            jax.ShapeDtypeStruct((N, 1, P2), jnp.int32),
            jax.ShapeDtypeStruct((N, D, P1), jnp.float32),
            jax.ShapeDtypeStruct((N, D, P2), jnp.float32),
        ],
    )(x, y, xnT, ynT)
    partials = _sc_normals(xh.reshape(N, D * P1), yh.reshape(N, D * P2),
                           idxx.reshape(N, P1), idxy.reshape(N, P2))
    cham_normals = jnp.sum(partials) / (P1 * N)
    return (dist[0, 0], cham_normals)
